# baseline (speedup 1.0000x reference)
import functools

import jax
import jax.numpy as jnp
from jax import lax
from jax.experimental import pallas as pl
from jax.experimental.pallas import tpu as pltpu
from jax.experimental.pallas import tpu_sc as plsc

VOCAB = 1000000
HALF = VOCAB // 2
DIM = 32
NUM_CLASSES = 100
B = 16384
T = 200
NW = 32
RPW = B // NW
RB = 4
NBLK = RPW // RB
NCH = 5
CH = T // NCH
IPB = RB * T
LANES = 16


def _avg_body(xids_hbm, lens_hbm, emb_hbm, out_hbm,
              ids_a, ids_b, lens_v, rows_a, rows_b, out_v,
              gsem_a, gsem_b, isem):
    cc = lax.axis_index("c")
    ss = lax.axis_index("s")
    base = (ss * 2 + cc) * RPW
    pltpu.sync_copy(lens_hbm.at[pl.ds(base, RPW), :], lens_v)

    def ids_copy(ids_ref, blk):
        return pltpu.make_async_copy(
            xids_hbm.at[pl.ds(base + blk * RB, RB), :, :], ids_ref, isem)

    def row_gathers(idx_ref, rows_ref, sem_ref, blk, r, start):
        len_s = jnp.max(lens_v[blk * RB + r, :])
        for c in range(NCH):
            cp = pltpu.make_async_copy(
                emb_hbm.at[idx_ref.at[r, c]],
                rows_ref.at[r, pl.ds(c * CH, CH)],
                sem_ref.at[r])
            go = cp.start if start else cp.wait
            if c == 0:
                go()
            else:
                pl.when(c * CH < len_s)(go)

    def row_compute(rows_ref, blk, r):
        lsplat = lens_v[blk * RB + r, :]
        len_s = jnp.max(lsplat)
        nch8 = (len_s + 7) // 8

        def tchunk(i, acc):
            a0, a1 = acc
            tb = i * 8
            tbs = jnp.broadcast_to(tb, (LANES,))
            for j in range(8):
                t = tb + j
                m = (tbs + j) < lsplat
                v0 = rows_ref[r, t, pl.ds(0, LANES)]
                v1 = rows_ref[r, t, pl.ds(LANES, LANES)]
                a0 = a0 + jnp.where(m, v0, 0.0)
                a1 = a1 + jnp.where(m, v1, 0.0)
            return a0, a1

        zero = jnp.zeros((LANES,), jnp.float32)
        a0, a1 = lax.fori_loop(0, nch8, tchunk, (zero, zero))
        inv = 1.0 / jnp.maximum(lsplat.astype(jnp.float32), 1.0)
        out_v[r, pl.ds(0, LANES)] = a0 * inv
        out_v[r, pl.ds(LANES, LANES)] = a1 * inv

    def handle_block(k, s_ids, s_rows, s_sem, n_ids, n_rows, n_sem):
        @pl.when(k + 1 < NBLK)
        def _():
            ids_copy(n_ids, k + 1).wait()
            for r in range(RB):
                row_gathers(n_ids, n_rows, n_sem, k + 1, r, True)
        for r in range(RB):
            row_gathers(s_ids, s_rows, s_sem, k, r, False)
            row_compute(s_rows, k, r)
        @pl.when(k + 2 < NBLK)
        def _():
            ids_copy(s_ids, k + 2).start()
        pltpu.sync_copy(out_v, out_hbm.at[pl.ds(base + k * RB, RB), :])

    ic = ids_copy(ids_a, 0)
    ic.start()
    ic.wait()
    for r in range(RB):
        row_gathers(ids_a, rows_a, gsem_a, 0, r, True)
    ids_copy(ids_b, 1).start()

    def loop(j, carry):
        handle_block(2 * j, ids_a, rows_a, gsem_a, ids_b, rows_b, gsem_b)
        handle_block(2 * j + 1, ids_b, rows_b, gsem_b, ids_a, rows_a, gsem_a)
        return carry

    lax.fori_loop(0, NBLK // 2, loop, 0)


def _make_avg():
    mesh = plsc.VectorSubcoreMesh(core_axis_name="c", subcore_axis_name="s")
    return pl.kernel(
        _avg_body,
        mesh=mesh,
        out_type=jax.ShapeDtypeStruct((B, DIM), jnp.float32),
        scratch_types=[
            pltpu.VMEM((RB, NCH, CH), jnp.int32),
            pltpu.VMEM((RB, NCH, CH), jnp.int32),
            pltpu.VMEM((RPW, LANES), jnp.int32),
            pltpu.VMEM((RB, T, DIM), jnp.float32),
            pltpu.VMEM((RB, T, DIM), jnp.float32),
            pltpu.VMEM((RB, DIM), jnp.float32),
            pltpu.SemaphoreType.DMA((RB,)),
            pltpu.SemaphoreType.DMA((RB,)),
            pltpu.SemaphoreType.DMA,
        ],
        compiler_params=pltpu.CompilerParams(
            use_tc_tiling_on_sc=False, needs_layout_passes=False),
    )


def _mm_body(x_ref, y_ref, w_ref, b_ref, o_ref):
    o_ref[...] = (
        jnp.dot(x_ref[...] + y_ref[...], w_ref[...],
                preferred_element_type=jnp.float32)
        + b_ref[...]
    )


def _mm(x, y, w, b):
    return pl.pallas_call(
        _mm_body,
        grid=(16,),
        in_specs=[
            pl.BlockSpec((B // 16, DIM), lambda i: (i, 0)),
            pl.BlockSpec((B // 16, DIM), lambda i: (i, 0)),
            pl.BlockSpec((DIM, 128), lambda i: (0, 0)),
            pl.BlockSpec((1, 128), lambda i: (0, 0)),
        ],
        out_specs=pl.BlockSpec((B // 16, 128), lambda i: (i, 0)),
        out_shape=jax.ShapeDtypeStruct((B, 128), jnp.float32),
    )(x, y, w, b)


def kernel(x_ids, lengths, emb, fc_w, fc_b):
    ids = x_ids.astype(jnp.int32).reshape(B, NCH, CH)
    ids_lo = jnp.where(ids < HALF, ids, HALF)
    ids_hi = jnp.where(ids >= HALF, ids - HALF, HALF)
    lens_splat = jnp.broadcast_to(
        lengths.astype(jnp.int32)[:, None], (B, LANES))
    tab_lo = jnp.pad(lax.slice(emb, (0, 0), (HALF, DIM)), ((0, 1), (0, 0)))
    tab_hi = jnp.pad(lax.slice(emb, (HALF, 0), (VOCAB, DIM)), ((0, 1), (0, 0)))
    avg_lo = _make_avg()(ids_lo, lens_splat, tab_lo)
    avg_hi = _make_avg()(ids_hi, lens_splat, tab_hi)
    wp = jnp.zeros((DIM, 128), jnp.float32).at[:, :NUM_CLASSES].set(fc_w.T)
    bp = jnp.zeros((1, 128), jnp.float32).at[:, :NUM_CLASSES].set(fc_b)
    out = _mm(avg_lo, avg_hi, wp, bp)
    return out[:, :NUM_CLASSES]


# speedup vs baseline: 15.0563x; 15.0563x over previous
import functools

import jax
import jax.numpy as jnp
from jax import lax
from jax.experimental import pallas as pl
from jax.experimental.pallas import tpu as pltpu
from jax.experimental.pallas import tpu_sc as plsc

VOCAB = 1000000
HALF = VOCAB // 2
DIM = 32
NUM_CLASSES = 100
B = 16384
T = 200
NW = 32
RPW = B // NW
RB = 4
NBLK = RPW // RB
NCH = 5
CH = T // NCH
IPB = RB * T
LANES = 16


def _avg_body(xids_hbm, lens_hbm, emb_hbm, out_hbm,
              ids_a, ids_b, lens_v, rows_a, rows_b, out_v,
              gsem_a, gsem_b, isem):
    cc = lax.axis_index("c")
    ss = lax.axis_index("s")
    base = (ss * 2 + cc) * RPW
    pltpu.sync_copy(lens_hbm.at[pl.ds(base, RPW), :], lens_v)

    def ids_copy(ids_ref, blk):
        return pltpu.make_async_copy(
            xids_hbm.at[pl.ds(base + blk * RB, RB), :, :], ids_ref, isem)

    def row_gathers(idx_ref, rows_ref, sem_ref, blk, r, start):
        len_s = jnp.max(lens_v[blk * RB + r, :])
        for c in range(NCH):
            cp = pltpu.make_async_copy(
                emb_hbm.at[idx_ref.at[r, c]],
                rows_ref.at[r, pl.ds(c * CH, CH)],
                sem_ref.at[r])
            go = cp.start if start else cp.wait
            if c == 0:
                go()
            else:
                pl.when(c * CH < len_s)(go)

    def row_compute(rows_ref, blk, r):
        lsplat = lens_v[blk * RB + r, :]
        len_s = jnp.max(lsplat)
        nch8 = (len_s + 7) // 8

        def tchunk(i, acc):
            a0, a1 = acc
            tb = i * 8
            tbs = jnp.broadcast_to(tb, (LANES,))
            for j in range(8):
                t = tb + j
                m = (tbs + j) < lsplat
                v0 = rows_ref[r, t, pl.ds(0, LANES)]
                v1 = rows_ref[r, t, pl.ds(LANES, LANES)]
                a0 = a0 + jnp.where(m, v0, 0.0)
                a1 = a1 + jnp.where(m, v1, 0.0)
            return a0, a1

        zero = jnp.zeros((LANES,), jnp.float32)
        a0, a1 = lax.fori_loop(0, nch8, tchunk, (zero, zero))
        inv = 1.0 / jnp.maximum(lsplat.astype(jnp.float32), 1.0)
        out_v[r, pl.ds(0, LANES)] = a0 * inv
        out_v[r, pl.ds(LANES, LANES)] = a1 * inv

    def handle_block(k, s_ids, s_rows, s_sem, n_ids, n_rows, n_sem):
        @pl.when(k + 1 < NBLK)
        def _():
            ids_copy(n_ids, k + 1).wait()
            for r in range(RB):
                row_gathers(n_ids, n_rows, n_sem, k + 1, r, True)
        for r in range(RB):
            row_gathers(s_ids, s_rows, s_sem, k, r, False)
            row_compute(s_rows, k, r)
        @pl.when(k + 2 < NBLK)
        def _():
            ids_copy(s_ids, k + 2).start()
        pltpu.sync_copy(out_v, out_hbm.at[pl.ds(base + k * RB, RB), :])

    ic = ids_copy(ids_a, 0)
    ic.start()
    ic.wait()
    for r in range(RB):
        row_gathers(ids_a, rows_a, gsem_a, 0, r, True)
    ids_copy(ids_b, 1).start()

    def loop(j, carry):
        handle_block(2 * j, ids_a, rows_a, gsem_a, ids_b, rows_b, gsem_b)
        handle_block(2 * j + 1, ids_b, rows_b, gsem_b, ids_a, rows_a, gsem_a)
        return carry

    lax.fori_loop(0, NBLK // 2, loop, 0)


def _make_avg():
    mesh = plsc.VectorSubcoreMesh(core_axis_name="c", subcore_axis_name="s")
    return pl.kernel(
        _avg_body,
        mesh=mesh,
        out_type=jax.ShapeDtypeStruct((B, DIM), jnp.float32),
        scratch_types=[
            pltpu.VMEM((RB, NCH, CH), jnp.int32),
            pltpu.VMEM((RB, NCH, CH), jnp.int32),
            pltpu.VMEM((RPW, LANES), jnp.int32),
            pltpu.VMEM((RB, T, DIM), jnp.float32),
            pltpu.VMEM((RB, T, DIM), jnp.float32),
            pltpu.VMEM((RB, DIM), jnp.float32),
            pltpu.SemaphoreType.DMA((RB,)),
            pltpu.SemaphoreType.DMA((RB,)),
            pltpu.SemaphoreType.DMA,
        ],
        compiler_params=pltpu.CompilerParams(
            use_tc_tiling_on_sc=False, needs_layout_passes=False),
    )


def _mm_body(x_ref, y_ref, w_ref, b_ref, o_ref):
    o_ref[...] = (
        jnp.dot(x_ref[...] + y_ref[...], w_ref[...],
                preferred_element_type=jnp.float32)
        + b_ref[...]
    )


def _mm(x, y, w, b):
    return pl.pallas_call(
        _mm_body,
        grid=(16,),
        in_specs=[
            pl.BlockSpec((B // 16, DIM), lambda i: (i, 0)),
            pl.BlockSpec((B // 16, DIM), lambda i: (i, 0)),
            pl.BlockSpec((DIM, 128), lambda i: (0, 0)),
            pl.BlockSpec((1, 128), lambda i: (0, 0)),
        ],
        out_specs=pl.BlockSpec((B // 16, 128), lambda i: (i, 0)),
        out_shape=jax.ShapeDtypeStruct((B, 128), jnp.float32),
    )(x, y, w, b)


def kernel(x_ids, lengths, emb, fc_w, fc_b):
    ids = x_ids.astype(jnp.int32).reshape(B, NCH, CH)
    spread = HALF + (ids & 4095)
    ids_lo = jnp.where(ids < HALF, ids, spread)
    ids_hi = jnp.where(ids >= HALF, ids - HALF, spread)
    lens_splat = jnp.broadcast_to(
        lengths.astype(jnp.int32)[:, None], (B, LANES))
    tab_lo = jnp.pad(lax.slice(emb, (0, 0), (HALF, DIM)), ((0, 4096), (0, 0)))
    tab_hi = jnp.pad(lax.slice(emb, (HALF, 0), (VOCAB, DIM)), ((0, 4096), (0, 0)))
    avg_lo = _make_avg()(ids_lo, lens_splat, tab_lo)
    avg_hi = _make_avg()(ids_hi, lens_splat, tab_hi)
    wp = jnp.zeros((DIM, 128), jnp.float32).at[:, :NUM_CLASSES].set(fc_w.T)
    bp = jnp.zeros((1, 128), jnp.float32).at[:, :NUM_CLASSES].set(fc_b)
    out = _mm(avg_lo, avg_hi, wp, bp)
    return out[:, :NUM_CLASSES]


# speedup vs baseline: 25.9038x; 1.7205x over previous
import functools

import jax
import jax.numpy as jnp
from jax import lax
from jax.experimental import pallas as pl
from jax.experimental.pallas import tpu as pltpu
from jax.experimental.pallas import tpu_sc as plsc

DIM = 32
NUM_CLASSES = 100
B = 16384
T = 200
NW = 32
RPW = B // NW
RB = 4
NBLK = RPW // RB
NCH = 4
CH = T // NCH
LANES = 16


def _avg_body(xids_hbm, lens_hbm, emb_hbm, out_hbm,
              ids_a, ids_b, lens_v, rows_a, rows_b, out_v,
              gsem_a, gsem_b, isem):
    cc = lax.axis_index("c")
    ss = lax.axis_index("s")
    base = (ss * 2 + cc) * RPW
    pltpu.sync_copy(lens_hbm.at[pl.ds(base, RPW), :], lens_v)

    def ids_copy(ids_ref, blk):
        return pltpu.make_async_copy(
            xids_hbm.at[pl.ds(base + blk * RB, RB), :, :], ids_ref, isem)

    def row_gathers(ids_ref, rows_ref, sem_ref, blk, r, start):
        len_s = jnp.max(lens_v[blk * RB + r, :])
        for c in range(NCH):
            cp = pltpu.make_async_copy(
                emb_hbm.at[ids_ref.at[r, c]],
                rows_ref.at[r, pl.ds(c * CH, CH)],
                sem_ref.at[r])
            go = cp.start if start else cp.wait
            if c == 0:
                go()
            else:
                pl.when(c * CH < len_s)(go)

    def row_compute(rows_ref, blk, r):
        lsplat = lens_v[blk * RB + r, :]
        len_s = jnp.max(lsplat)
        nch8 = (len_s + 7) // 8

        def tchunk(i, acc):
            a0, a1 = acc
            tb = i * 8
            tbs = jnp.broadcast_to(tb, (LANES,))
            for j in range(8):
                t = tb + j
                m = (tbs + j) < lsplat
                v0 = rows_ref[r, t, pl.ds(0, LANES)]
                v1 = rows_ref[r, t, pl.ds(LANES, LANES)]
                a0 = a0 + jnp.where(m, v0, 0.0)
                a1 = a1 + jnp.where(m, v1, 0.0)
            return a0, a1

        zero = jnp.zeros((LANES,), jnp.float32)
        a0, a1 = lax.fori_loop(0, nch8, tchunk, (zero, zero))
        inv = 1.0 / jnp.maximum(lsplat.astype(jnp.float32), 1.0)
        out_v[r, pl.ds(0, LANES)] = a0 * inv
        out_v[r, pl.ds(LANES, LANES)] = a1 * inv

    def handle_block(k, s_ids, s_rows, s_sem, n_ids, n_rows, n_sem):
        @pl.when(k + 1 < NBLK)
        def _():
            ids_copy(n_ids, k + 1).wait()
            for r in range(RB):
                row_gathers(n_ids, n_rows, n_sem, k + 1, r, True)
        for r in range(RB):
            row_gathers(s_ids, s_rows, s_sem, k, r, False)
            row_compute(s_rows, k, r)
        @pl.when(k + 2 < NBLK)
        def _():
            ids_copy(s_ids, k + 2).start()
        pltpu.sync_copy(out_v, out_hbm.at[pl.ds(base + k * RB, RB), :])

    ic = ids_copy(ids_a, 0)
    ic.start()
    ic.wait()
    for r in range(RB):
        row_gathers(ids_a, rows_a, gsem_a, 0, r, True)
    ids_copy(ids_b, 1).start()

    def loop(j, carry):
        handle_block(2 * j, ids_a, rows_a, gsem_a, ids_b, rows_b, gsem_b)
        handle_block(2 * j + 1, ids_b, rows_b, gsem_b, ids_a, rows_a, gsem_a)
        return carry

    lax.fori_loop(0, NBLK // 2, loop, 0)


def _make_avg():
    mesh = plsc.VectorSubcoreMesh(core_axis_name="c", subcore_axis_name="s")
    return pl.kernel(
        _avg_body,
        mesh=mesh,
        out_type=jax.ShapeDtypeStruct((B, DIM), jnp.float32),
        scratch_types=[
            pltpu.VMEM((RB, NCH, CH), jnp.int32),
            pltpu.VMEM((RB, NCH, CH), jnp.int32),
            pltpu.VMEM((RPW, LANES), jnp.int32),
            pltpu.VMEM((RB, T, DIM), jnp.float32),
            pltpu.VMEM((RB, T, DIM), jnp.float32),
            pltpu.VMEM((RB, DIM), jnp.float32),
            pltpu.SemaphoreType.DMA((RB,)),
            pltpu.SemaphoreType.DMA((RB,)),
            pltpu.SemaphoreType.DMA,
        ],
        compiler_params=pltpu.CompilerParams(
            use_tc_tiling_on_sc=False, needs_layout_passes=False),
    )


def _mm_body(x_ref, w_ref, b_ref, o_ref):
    o_ref[...] = (
        jnp.dot(x_ref[...], w_ref[...], preferred_element_type=jnp.float32)
        + b_ref[...]
    )


def _mm(x, w, b):
    return pl.pallas_call(
        _mm_body,
        grid=(16,),
        in_specs=[
            pl.BlockSpec((B // 16, DIM), lambda i: (i, 0)),
            pl.BlockSpec((DIM, 128), lambda i: (0, 0)),
            pl.BlockSpec((1, 128), lambda i: (0, 0)),
        ],
        out_specs=pl.BlockSpec((B // 16, 128), lambda i: (i, 0)),
        out_shape=jax.ShapeDtypeStruct((B, 128), jnp.float32),
    )(x, w, b)


def kernel(x_ids, lengths, emb, fc_w, fc_b):
    x_ids = x_ids.astype(jnp.int32).reshape(B, NCH, CH)
    lens_splat = jnp.broadcast_to(
        lengths.astype(jnp.int32)[:, None], (B, LANES))
    avg = _make_avg()(x_ids, lens_splat, emb)
    wp = jnp.zeros((DIM, 128), jnp.float32).at[:, :NUM_CLASSES].set(fc_w.T)
    bp = jnp.zeros((1, 128), jnp.float32).at[:, :NUM_CLASSES].set(fc_b)
    out = _mm(avg, wp, bp)
    return out[:, :NUM_CLASSES]
